# baseline (speedup 1.0000x reference)
import functools

import jax
import jax.numpy as jnp
from jax import lax
from jax.experimental import pallas as pl
from jax.experimental.pallas import tpu as pltpu
from jax.experimental.pallas import tpu_sc as plsc

B = 4
L = 8192
D = 768
CHUNK = 64
SCH = 64


@functools.cache
def _build_sc_kernel():
    info = plsc.get_sparse_core_info()
    nw = info.num_cores * info.num_subcores
    ns = info.num_subcores
    rows_per_w = L // nw
    n_chunks = rows_per_w // CHUNK

    mesh = plsc.VectorSubcoreMesh(core_axis_name="c", subcore_axis_name="s")

    @functools.partial(
        pl.kernel,
        mesh=mesh,
        out_type=jax.ShapeDtypeStruct((B, L, D), jnp.float32),
        scratch_types=[
            pltpu.VMEM((CHUNK, D), jnp.float32),
            pltpu.VMEM_SHARED((ns, SCH, D), jnp.float32),
            pltpu.SemaphoreType.DMA,
            pltpu.SemaphoreType.DMA,
            pltpu.SemaphoreType.DMA,
            pltpu.SemaphoreType.DMA,
        ],
    )
    def k(emb_hbm, out_hbm, tbuf, sbuf, lsa, lsb, sas, sbs):
        sid = lax.axis_index("s")
        cid = lax.axis_index("c")
        wid = sid * info.num_cores + cid
        base = wid * rows_per_w
        slot = sbuf.at[sid]
        for c in range(n_chunks):
            row = base + c * CHUNK
            la = pltpu.make_async_copy(emb_hbm.at[pl.ds(row, CHUNK)], tbuf, lsa)
            la.start()
            a_stores = []
            b_stores = []
            for h in range(CHUNK // SCH):
                hrow = row + h * SCH
                lb = pltpu.make_async_copy(emb_hbm.at[pl.ds(hrow, SCH)], slot, lsb)
                lb.start()
                if h == 0:
                    la.wait()
                    for b in (0, 1):
                        cp = pltpu.make_async_copy(
                            tbuf, out_hbm.at[b, pl.ds(row, CHUNK)], sas
                        )
                        cp.start()
                        a_stores.append(cp)
                lb.wait()
                for b in (2, 3):
                    cp = pltpu.make_async_copy(
                        slot, out_hbm.at[b, pl.ds(hrow, SCH)], sbs
                    )
                    cp.start()
                    b_stores.append(cp)
                for cp in b_stores:
                    cp.wait()
                b_stores = []
            for cp in a_stores:
                cp.wait()

    return k


def kernel(x, Embed):
    return _build_sc_kernel()(Embed)


# speedup vs baseline: 1.2234x; 1.2234x over previous
import functools

import jax
import jax.numpy as jnp
from jax import lax
from jax.experimental import pallas as pl
from jax.experimental.pallas import tpu as pltpu
from jax.experimental.pallas import tpu_sc as plsc

B = 4
L = 8192
D = 768
CHUNK = 128


@functools.cache
def _build_sc_kernel():
    info = plsc.get_sparse_core_info()
    nw = info.num_cores * info.num_subcores
    rows_per_w = L // nw
    n_chunks = rows_per_w // CHUNK

    mesh = plsc.VectorSubcoreMesh(core_axis_name="c", subcore_axis_name="s")

    @functools.partial(
        pl.kernel,
        mesh=mesh,
        out_type=jax.ShapeDtypeStruct((B, L, D), jnp.float32),
        scratch_types=[pltpu.VMEM((CHUNK, D), jnp.float32)],
    )
    def k(emb_hbm, out_hbm, buf):
        wid = lax.axis_index("s") * info.num_cores + lax.axis_index("c")
        base = wid * rows_per_w
        for c in range(n_chunks):
            row = base + c * CHUNK
            pltpu.sync_copy(emb_hbm.at[pl.ds(row, CHUNK)], buf)
            for b in range(B):
                pltpu.sync_copy(buf, out_hbm.at[b, pl.ds(row, CHUNK)])

    return k


def kernel(x, Embed):
    return _build_sc_kernel()(Embed)
